# baseline (speedup 1.0000x reference)
import jax
import jax.numpy as jnp
from jax import lax
from jax.experimental import pallas as pl
from jax.experimental.pallas import tpu as pltpu
from jax.experimental.pallas import tpu_sc as plsc

_SEQ = 8192
_DIM = 1024
_NC = 2
_ROWS = _SEQ // _NC
_CH = 512
_NCH = _ROWS // _CH
_NBUF = 3


def _scs_body(emb_hbm, out_hbm, buf, *sems):
    gsem = sems[:_NBUF]
    ssem = sems[_NBUF:]
    cid = lax.axis_index("c")
    base = cid * _ROWS

    gat = [None] * _NBUF
    sca = [None] * _NBUF

    def start_gather(i):
        b = i % _NBUF
        gat[b] = pltpu.async_copy(
            emb_hbm.at[pl.ds(base + i * _CH, _CH)], buf.at[b], gsem[b])

    for i in range(min(_NBUF, _NCH)):
        start_gather(i)
    for i in range(_NCH):
        b = i % _NBUF
        gat[b].wait()
        sca[b] = pltpu.async_copy(
            buf.at[b], out_hbm.at[pl.ds(base + i * _CH, _CH)], ssem[b])
        nxt = i + _NBUF
        if nxt < _NCH:
            sca[b].wait()
            sca[b] = None
            start_gather(nxt)
    for b in range(_NBUF):
        if sca[b] is not None:
            sca[b].wait()


@jax.jit
def _sc_copy(emb):
    mesh = plsc.ScalarSubcoreMesh(axis_name="c", num_cores=2)
    fn = pl.kernel(
        _scs_body,
        mesh=mesh,
        out_type=jax.ShapeDtypeStruct((_SEQ, _DIM), jnp.float32),
        scratch_types=(
            [pltpu.VMEM_SHARED((_NBUF, _CH, _DIM), jnp.float32)]
            + [pltpu.SemaphoreType.DMA] * (2 * _NBUF)
        ),
    )
    return fn(emb)


def kernel(x, emb):
    return _sc_copy(emb)[None]


# speedup vs baseline: 1.0725x; 1.0725x over previous
import jax
import jax.numpy as jnp
from jax import lax
from jax.experimental import pallas as pl
from jax.experimental.pallas import tpu as pltpu
from jax.experimental.pallas import tpu_sc as plsc

_SEQ = 8192
_DIM = 1024
_NW = 32
_ROWS = _SEQ // _NW
_CH = 32
_NCH = _ROWS // _CH
_NBUF = 3


def _sc_body(emb_hbm, out_hbm, buf, *sems):
    gsem = sems[:_NBUF]
    ssem = sems[_NBUF:]
    wid = lax.axis_index("s") * 2 + lax.axis_index("c")
    base = wid * _ROWS

    gat = [None] * _NBUF
    sca = [None] * _NBUF

    def start_gather(i):
        b = i % _NBUF
        gat[b] = pltpu.async_copy(
            emb_hbm.at[pl.ds(base + i * _CH, _CH)], buf.at[b], gsem[b])

    for i in range(min(_NBUF, _NCH)):
        start_gather(i)
    for i in range(_NCH):
        b = i % _NBUF
        gat[b].wait()
        sca[b] = pltpu.async_copy(
            buf.at[b], out_hbm.at[pl.ds(base + i * _CH, _CH)], ssem[b])
        nxt = i + _NBUF
        if nxt < _NCH:
            sca[b].wait()
            sca[b] = None
            start_gather(nxt)
    for b in range(_NBUF):
        if sca[b] is not None:
            sca[b].wait()


@jax.jit
def _sc_copy(emb):
    mesh = plsc.VectorSubcoreMesh(core_axis_name="c", subcore_axis_name="s")
    fn = pl.kernel(
        _sc_body,
        mesh=mesh,
        out_type=jax.ShapeDtypeStruct((_SEQ, _DIM), jnp.float32),
        scratch_types=(
            [pltpu.VMEM((_NBUF, _CH, _DIM), jnp.float32)]
            + [pltpu.SemaphoreType.DMA] * (2 * _NBUF)
        ),
    )
    return fn(emb)


def kernel(x, emb):
    return _sc_copy(emb)[None]


# speedup vs baseline: 1.0796x; 1.0067x over previous
import functools

import jax
import jax.numpy as jnp
from jax import lax
from jax.experimental import pallas as pl
from jax.experimental.pallas import tpu as pltpu
from jax.experimental.pallas import tpu_sc as plsc

_NW = 32
_CH = 32
_NBUF = 3


@functools.lru_cache(maxsize=None)
def _build(seq, dim, dtype_name):
    dtype = jnp.dtype(dtype_name)
    rows = seq // _NW
    nch = rows // _CH

    def _sc_body(emb_hbm, out_hbm, buf, *sems):
        gsem = sems[:_NBUF]
        ssem = sems[_NBUF:]
        wid = lax.axis_index("s") * 2 + lax.axis_index("c")
        base = wid * rows

        gat = [None] * _NBUF
        sca = [None] * _NBUF

        def start_gather(i):
            b = i % _NBUF
            gat[b] = pltpu.async_copy(
                emb_hbm.at[pl.ds(base + i * _CH, _CH)], buf.at[b], gsem[b])

        for i in range(min(_NBUF, nch)):
            start_gather(i)
        for i in range(nch):
            b = i % _NBUF
            gat[b].wait()
            sca[b] = pltpu.async_copy(
                buf.at[b], out_hbm.at[pl.ds(base + i * _CH, _CH)], ssem[b])
            nxt = i + _NBUF
            if nxt < nch:
                sca[b].wait()
                sca[b] = None
                start_gather(nxt)
        for b in range(_NBUF):
            if sca[b] is not None:
                sca[b].wait()

    mesh = plsc.VectorSubcoreMesh(core_axis_name="c", subcore_axis_name="s")
    return pl.kernel(
        _sc_body,
        mesh=mesh,
        out_type=jax.ShapeDtypeStruct((seq, dim), dtype),
        scratch_types=(
            [pltpu.VMEM((_NBUF, _CH, dim), dtype)]
            + [pltpu.SemaphoreType.DMA] * (2 * _NBUF)
        ),
    )


def kernel(x, emb):
    seq = x.shape[1]
    fn = _build(seq, emb.shape[1], emb.dtype.name)
    return fn(emb)[None]
